# Initial kernel scaffold; baseline (speedup 1.0000x reference)
import jax
import jax.numpy as jnp
from jax.experimental import pallas as pl


def kernel(h_v, h_w, e_vw, edge_matrix):
    raise NotImplementedError("write your pallas kernel here")


# speedup vs baseline: 2.5746x; 2.5746x over previous
import jax
import jax.numpy as jnp
from jax.experimental import pallas as pl
from jax.experimental.pallas import tpu as pltpu

_N_LABELS = 4
_BLOCK_B = 200


def _ggnn_body(e_ref, x_ref, w_ref, o_ref):
    x = x_ref[...]
    b = x.shape[0]
    x2 = x.reshape(b * 32, 128)
    e = e_ref[...]
    out = None
    for i in range(_N_LABELS):
        p = jax.lax.dot_general(
            x2, w_ref[i],
            dimension_numbers=(((1,), (0,)), ((), ())),
            preferred_element_type=jnp.float32,
        ).reshape(b, 32, 128)
        m = (e == i).astype(jnp.float32)
        t = m * p
        out = t if out is None else out + t
    o_ref[...] = out


def kernel(h_v, h_w, e_vw, edge_matrix):
    del h_v
    nb, nn, nin = h_w.shape
    nout = edge_matrix.shape[1]
    bb = _BLOCK_B
    grid = (nb // bb,)
    em_t = jnp.transpose(edge_matrix, (0, 2, 1))
    return pl.pallas_call(
        _ggnn_body,
        grid=grid,
        in_specs=[
            pl.BlockSpec((bb, nn, 1), lambda g: (g, 0, 0)),
            pl.BlockSpec((bb, nn, nin), lambda g: (g, 0, 0)),
            pl.BlockSpec((_N_LABELS, nin, nout), lambda g: (0, 0, 0)),
        ],
        out_specs=pl.BlockSpec((bb, nn, nout), lambda g: (g, 0, 0)),
        out_shape=jax.ShapeDtypeStruct((nb, nn, nout), jnp.float32),
        compiler_params=pltpu.CompilerParams(
            dimension_semantics=("arbitrary",),
        ),
    )(e_vw, h_w, em_t)


# speedup vs baseline: 5.3647x; 2.0837x over previous
import jax
import jax.numpy as jnp
from jax.experimental import pallas as pl
from jax.experimental.pallas import tpu as pltpu

_N_LABELS = 4
_BLOCK_B = 400


def _ggnn_body(e_ref, x_ref, w_ref, o_ref):
    x = x_ref[...]
    b = x.shape[0]
    x2 = x.reshape(b * 32, 128)
    e = e_ref[...][:, :, None]
    out = None
    for i in range(_N_LABELS):
        p = jax.lax.dot_general(
            x2, w_ref[i],
            dimension_numbers=(((1,), (0,)), ((), ())),
            preferred_element_type=jnp.float32,
        ).reshape(b, 32, 128)
        m = (e == i).astype(jnp.float32)
        t = m * p
        out = t if out is None else out + t
    o_ref[...] = out


def kernel(h_v, h_w, e_vw, edge_matrix):
    del h_v
    nb, nn, nin = h_w.shape
    nout = edge_matrix.shape[1]
    bb = _BLOCK_B
    grid = (nb // bb,)
    em_t = jnp.transpose(edge_matrix, (0, 2, 1))
    e2 = jnp.reshape(e_vw, (nb, nn))
    return pl.pallas_call(
        _ggnn_body,
        grid=grid,
        in_specs=[
            pl.BlockSpec((bb, nn), lambda g: (g, 0)),
            pl.BlockSpec((bb, nn, nin), lambda g: (g, 0, 0)),
            pl.BlockSpec((_N_LABELS, nin, nout), lambda g: (0, 0, 0)),
        ],
        out_specs=pl.BlockSpec((bb, nn, nout), lambda g: (g, 0, 0)),
        out_shape=jax.ShapeDtypeStruct((nb, nn, nout), jnp.float32),
        compiler_params=pltpu.CompilerParams(
            dimension_semantics=("arbitrary",),
        ),
    )(e2, h_w, em_t)
